# baseline (speedup 1.0000x reference)
import functools

import jax
import jax.numpy as jnp
from jax import lax
from jax.experimental import pallas as pl
from jax.experimental.pallas import tpu as pltpu
from jax.experimental.pallas import tpu_sc as plsc

D = 64


@functools.lru_cache(maxsize=None)
def _make_gather(BATCH: int, HIST: int, NBUF: int):
    info = plsc.get_sparse_core_info()
    NC, NS = info.num_cores, info.num_subcores
    NW = NC * NS
    i_per_w = BATCH // NW
    assert i_per_w * NW == BATCH and i_per_w % NBUF == 0

    mesh = plsc.VectorSubcoreMesh(core_axis_name="c", subcore_axis_name="s")

    @functools.partial(
        pl.kernel,
        mesh=mesh,
        compiler_params=pltpu.CompilerParams(use_tc_tiling_on_sc=False),
        out_type=jax.ShapeDtypeStruct((BATCH, HIST, D), jnp.float32),
        scratch_types=[
            pltpu.VMEM((i_per_w, HIST), jnp.int32),
            pltpu.VMEM((NBUF, HIST, D), jnp.float32),
            pltpu.SemaphoreType.DMA((NBUF,)),
            pltpu.SemaphoreType.DMA((NBUF,)),
        ],
    )
    def gather_kernel(idx_hbm, table_hbm, out_hbm, idx_v, rows_v, sem_g, sem_o):
        wid = lax.axis_index("s") * NC + lax.axis_index("c")
        i_base = wid * i_per_w

        pltpu.sync_copy(idx_hbm.at[pl.ds(i_base, i_per_w), :], idx_v)

        def gather(g, b):
            pltpu.async_copy(
                table_hbm.at[idx_v.at[g]], rows_v.at[b], sem_g.at[b]
            )

        def wait_gather(g, b):
            pltpu.make_async_copy(
                table_hbm.at[idx_v.at[g]], rows_v.at[b], sem_g.at[b]
            ).wait()

        def writeback(g, b):
            pltpu.async_copy(rows_v.at[b], out_hbm.at[i_base + g], sem_o.at[b])

        def wait_writeback(g, b):
            pltpu.make_async_copy(
                rows_v.at[b], out_hbm.at[i_base + g], sem_o.at[b]
            ).wait()

        for b in range(NBUF):
            gather(b, b)

        def body(s, carry):
            g0 = s * NBUF
            for b in range(NBUF):
                wait_gather(g0 + b, b)
                writeback(g0 + b, b)
            for b in range(NBUF):
                wait_writeback(g0 + b, b)
                gather(g0 + NBUF + b, b)
            return carry

        n_passes = i_per_w // NBUF
        lax.fori_loop(0, n_passes - 1, body, 0)

        g0 = (n_passes - 1) * NBUF
        for b in range(NBUF):
            wait_gather(g0 + b, b)
            writeback(g0 + b, b)
        for b in range(NBUF):
            wait_writeback(g0 + b, b)

    return gather_kernel


def kernel(input_ids, table):
    batch, hist = input_ids.shape
    return _make_gather(batch, hist, 4)(input_ids, table)


# speedup vs baseline: 1.3610x; 1.3610x over previous
import functools

import jax
import jax.numpy as jnp
from jax import lax
from jax.experimental import pallas as pl
from jax.experimental.pallas import tpu as pltpu
from jax.experimental.pallas import tpu_sc as plsc

D = 64
HIST = 50
LANES = 128
HPAD = 56


@functools.lru_cache(maxsize=None)
def _make_gather(BATCH: int, NBUF: int):
    info = plsc.get_sparse_core_info()
    NC, NS = info.num_cores, info.num_subcores
    NW = NC * NS
    i_per_w = BATCH // NW
    assert i_per_w * NW == BATCH and i_per_w % NBUF == 0
    n_stage = i_per_w * LANES

    mesh = plsc.VectorSubcoreMesh(core_axis_name="c", subcore_axis_name="s")

    @functools.partial(
        pl.kernel,
        mesh=mesh,
        compiler_params=pltpu.CompilerParams(use_tc_tiling_on_sc=False),
        out_type=jax.ShapeDtypeStruct((BATCH, HPAD, LANES), jnp.float32),
        scratch_types=[
            pltpu.VMEM((n_stage,), jnp.int32),
            pltpu.VMEM((NBUF, HPAD, D), jnp.float32),
            pltpu.SemaphoreType.DMA((NBUF,)),
            pltpu.SemaphoreType.DMA((NBUF,)),
        ],
    )
    def gather_kernel(idx_hbm, table_hbm, out_hbm, idx_v, rows_v, sem_g, sem_o):
        wid = lax.axis_index("s") * NC + lax.axis_index("c")
        i_base = wid * i_per_w

        pltpu.sync_copy(idx_hbm.at[pl.ds(i_base * LANES, n_stage)], idx_v)

        def gather(g, b):
            pltpu.async_copy(
                table_hbm.at[idx_v.at[pl.ds(g * LANES, HPAD)]],
                rows_v.at[b],
                sem_g.at[b],
            )

        def wait_gather(g, b):
            pltpu.make_async_copy(
                table_hbm.at[idx_v.at[pl.ds(g * LANES, HPAD)]],
                rows_v.at[b],
                sem_g.at[b],
            ).wait()

        def writeback(g, b):
            pltpu.async_copy(
                rows_v.at[b],
                out_hbm.at[i_base + g, :, pl.ds(0, D)],
                sem_o.at[b],
            )

        def wait_writeback(g, b):
            pltpu.make_async_copy(
                rows_v.at[b],
                out_hbm.at[i_base + g, :, pl.ds(0, D)],
                sem_o.at[b],
            ).wait()

        for b in range(NBUF):
            gather(b, b)

        def body(s, carry):
            g0 = s * NBUF
            for b in range(NBUF):
                wait_gather(g0 + b, b)
                writeback(g0 + b, b)
            for b in range(NBUF):
                wait_writeback(g0 + b, b)
                gather(g0 + NBUF + b, b)
            return carry

        n_passes = i_per_w // NBUF
        lax.fori_loop(0, n_passes - 1, body, 0)

        g0 = (n_passes - 1) * NBUF
        for b in range(NBUF):
            wait_gather(g0 + b, b)
            writeback(g0 + b, b)
        for b in range(NBUF):
            wait_writeback(g0 + b, b)

    return gather_kernel


def kernel(input_ids, table):
    batch, hist = input_ids.shape
    n_rows = table.shape[0]
    fill = (
        jnp.arange(batch, dtype=jnp.int32)[:, None] * (LANES - hist)
        + jnp.arange(LANES - hist, dtype=jnp.int32)[None, :]
    ) % n_rows
    ids128 = jnp.concatenate([input_ids, fill], axis=1)
    out_p = _make_gather(batch, 8)(ids128.reshape(-1), table)
    return out_p[:, :hist, :D]
